# Initial kernel scaffold; baseline (speedup 1.0000x reference)
import functools

import jax
import jax.numpy as jnp
from jax import lax
from jax.experimental import pallas as pl
from jax.experimental.pallas import tpu as pltpu
from jax.experimental.pallas import tpu_sc as plsc

N_CELL = 1000
NPAD = 1008
NCH = NPAD // 16
OFF = 112
WORK = OFF + NPAD + 112
BIGP = 1 << 29
MAXEV = 100
NSLICE = 16


def _nms_sc(xr):
    mesh = plsc.VectorSubcoreMesh(core_axis_name="c", subcore_axis_name="s")

    @functools.partial(
        pl.kernel,
        out_type=jax.ShapeDtypeStruct((NSLICE, 2, N_CELL), jnp.float32),
        mesh=mesh,
        scratch_types=[
            pltpu.VMEM((NPAD,), jnp.float32),
            pltpu.VMEM((NPAD,), jnp.float32),
            pltpu.VMEM((WORK,), jnp.float32),
            pltpu.VMEM((WORK,), jnp.int32),
            pltpu.VMEM((WORK,), jnp.float32),
            pltpu.VMEM((WORK,), jnp.int32),
            pltpu.VMEM((WORK,), jnp.float32),
            pltpu.VMEM((WORK,), jnp.int32),
            pltpu.VMEM((NPAD,), jnp.int32),
            pltpu.VMEM((NPAD,), jnp.int32),
            pltpu.VMEM((NPAD,), jnp.int32),
            pltpu.VMEM((NPAD,), jnp.int32),
            pltpu.VMEM((NPAD,), jnp.int32),
            pltpu.VMEM((NPAD,), jnp.int32),
            pltpu.VMEM((NPAD,), jnp.int32),
        ],
    )
    def k(x_hbm, out_hbm, e_in, c1_in, ae, ap, be, bp, ce, cp,
          eb, mzl, mzh, myl, myh, mxl, mxh):
        wid = lax.axis_index("s") * 2 + lax.axis_index("c")

        @pl.when(wid < NSLICE)
        def _():
            pltpu.sync_copy(x_hbm.at[wid, 0], e_in.at[pl.ds(0, N_CELL)])
            pltpu.sync_copy(x_hbm.at[wid, 1], c1_in.at[pl.ds(0, N_CELL)])

            def init_body(j, cnt):
                ids = j * 16 + lax.iota(jnp.int32, 16)
                sl = pl.ds(j * 16, 16)
                wsl = pl.ds(OFF + j * 16, 16)
                e = e_in[sl]
                valid = jnp.logical_and(ids < N_CELL, e != 0.0)
                ae[wsl] = jnp.where(valid, e, -1.0)
                ap[wsl] = jnp.where(valid, ids * 2, BIGP)
                eb[sl] = jnp.where(valid, plsc.bitcast(e, jnp.int32), -1)
                z = lax.rem(ids, 10)
                y = lax.rem(lax.div(ids, 10), 10)
                xx = lax.div(ids, 100)
                one = jnp.full((16,), 1, jnp.int32)
                zero = jnp.full((16,), 0, jnp.int32)
                mzl[sl] = jnp.where(z > 0, one, zero)
                mzh[sl] = jnp.where(z < 9, one, zero)
                myl[sl] = jnp.where(y > 0, one, zero)
                myh[sl] = jnp.where(y < 9, one, zero)
                mxl[sl] = jnp.where(xx > 0, one, zero)
                mxh[sl] = jnp.where(xx < 9, one, zero)
                return cnt + jnp.sum(jnp.where(valid, one, zero))

            und0 = lax.fori_loop(0, NCH, init_body, jnp.int32(0))

            def pool(src_e, src_p, dst_e, dst_p, stride, ml_ref, mh_ref):
                def body(j, carry):
                    b = OFF + j * 16
                    sl = pl.ds(j * 16, 16)
                    ce_ = src_e[pl.ds(b, 16)]
                    cp_ = src_p[pl.ds(b, 16)]
                    le = src_e[pl.ds(b - stride, 16)]
                    lp = src_p[pl.ds(b - stride, 16)]
                    re_ = src_e[pl.ds(b + stride, 16)]
                    rp = src_p[pl.ds(b + stride, 16)]
                    mlo = ml_ref[sl] != 0
                    mhi = mh_ref[sl] != 0
                    le = jnp.where(mlo, le, -2.0)
                    lp = jnp.where(mlo, lp, BIGP)
                    re_ = jnp.where(mhi, re_, -2.0)
                    rp = jnp.where(mhi, rp, BIGP)
                    gt = jnp.logical_or(
                        ce_ > le, jnp.logical_and(ce_ == le, cp_ < lp))
                    me = jnp.where(gt, ce_, le)
                    mp = jnp.where(gt, cp_, lp)
                    gt2 = jnp.logical_or(
                        me > re_, jnp.logical_and(me == re_, mp < rp))
                    dst_e[pl.ds(b, 16)] = jnp.where(gt2, me, re_)
                    dst_p[pl.ds(b, 16)] = jnp.where(gt2, mp, rp)
                    return carry
                lax.fori_loop(0, NCH, body, jnp.int32(0))

            def round_body(_c):
                pool(ae, ap, be, bp, 1, mzl, mzh)
                pool(be, bp, ce, cp, 10, myl, myh)
                pool(ce, cp, be, bp, 100, mxl, mxh)

                def upd(j, cnt):
                    b = pl.ds(OFF + j * 16, 16)
                    me = be[b]
                    mp = bp[b]
                    e = ae[b]
                    p = ap[b]
                    kb = (p & 1) == 1
                    und = jnp.logical_and(e > 0.0, jnp.logical_not(kb))
                    isself = jnp.logical_and(me == e, mp == p)
                    newk = jnp.logical_and(und, isself)
                    sup = und & ((mp & 1) == 1) & jnp.logical_not(isself)
                    ap[b] = jnp.where(newk, p | 1, p)
                    ae[b] = jnp.where(sup, -1.0, e)
                    rem = und & jnp.logical_not(newk) & jnp.logical_not(sup)
                    one = jnp.full((16,), 1, jnp.int32)
                    zero = jnp.full((16,), 0, jnp.int32)
                    return cnt + jnp.sum(jnp.where(rem, one, zero))

                return lax.fori_loop(0, NCH, upd, jnp.int32(0))

            lax.while_loop(lambda c: c > 0, round_body, und0)

            def kcount(j, cnt):
                p = ap[pl.ds(OFF + j * 16, 16)]
                return cnt + jnp.sum(p & 1)

            kept_n = lax.fori_loop(0, NCH, kcount, jnp.int32(0))

            @pl.when(kept_n > MAXEV)
            def _cap():
                one = jnp.full((16,), 1, jnp.int32)
                zero = jnp.full((16,), 0, jnp.int32)

                def count_gt(t):
                    def cb(j, cnt):
                        ebj = eb[pl.ds(j * 16, 16)]
                        return cnt + jnp.sum(jnp.where(ebj > t, one, zero))
                    return lax.fori_loop(0, NCH, cb, jnp.int32(0))

                def bs(_i, lohi):
                    lo, hi = lohi
                    mid = lax.div(lo + hi, jnp.int32(2))
                    pred = count_gt(mid) < MAXEV
                    return (jnp.where(pred, lo, mid + 1),
                            jnp.where(pred, mid, hi))

                tau, _ = lax.fori_loop(
                    0, 30, bs, (jnp.int32(0), jnp.int32((1 << 30) - 1)))
                quota = MAXEV - count_gt(tau)

                def capb(j, carry):
                    b = pl.ds(OFF + j * 16, 16)
                    sl = pl.ds(j * 16, 16)
                    ebj = eb[sl]
                    tie = ebj == tau
                    tc = jnp.where(tie, one, zero)
                    pfx = plsc.cumsum(tc)
                    surv = tie & ((carry + (pfx - tc)) < quota)
                    allow = (ebj > tau) | surv
                    p = ap[b]
                    ap[b] = jnp.where(allow, p, p & (~1))
                    return carry + jnp.max(pfx)

                lax.fori_loop(0, NCH, capb, jnp.int32(0))

            anyv = und0 > 0

            def ob(j, carry):
                b = pl.ds(OFF + j * 16, 16)
                sl = pl.ds(j * 16, 16)
                keepm = jnp.logical_or((ap[b] & 1) == 1,
                                       jnp.logical_not(anyv))
                e_in[sl] = jnp.where(keepm, e_in[sl], 0.0)
                c1_in[sl] = jnp.where(keepm, c1_in[sl], 0.0)
                return carry

            lax.fori_loop(0, NCH, ob, jnp.int32(0))
            pltpu.sync_copy(e_in.at[pl.ds(0, N_CELL)], out_hbm.at[wid, 0])
            pltpu.sync_copy(c1_in.at[pl.ds(0, N_CELL)], out_hbm.at[wid, 1])

    return k(xr)


def kernel(x):
    shape = x.shape
    xr = x.reshape(NSLICE, 2, N_CELL)
    out = _nms_sc(xr)
    return out.reshape(shape)


# speedup vs baseline: 43.3292x; 43.3292x over previous
import functools

import jax
import jax.numpy as jnp
from jax import lax
from jax.experimental import pallas as pl
from jax.experimental.pallas import tpu as pltpu
from jax.experimental.pallas import tpu_sc as plsc

N_CELL = 1000
NPAD = 1024
NCH = NPAD // 16
OFF = 112
WORK = OFF + NPAD + 112
BIGP = 1 << 29
MAXEV = 100
NSLICE = 16


def _nms_sc(xr):
    mesh = plsc.VectorSubcoreMesh(core_axis_name="c", subcore_axis_name="s")

    @functools.partial(
        pl.kernel,
        out_type=jax.ShapeDtypeStruct((NSLICE, 2, NPAD), jnp.float32),
        mesh=mesh,
        compiler_params=pltpu.CompilerParams(needs_layout_passes=False),
        scratch_types=[
            pltpu.VMEM((NPAD,), jnp.float32),
            pltpu.VMEM((NPAD,), jnp.float32),
            pltpu.VMEM((WORK,), jnp.float32),
            pltpu.VMEM((WORK,), jnp.int32),
            pltpu.VMEM((WORK,), jnp.float32),
            pltpu.VMEM((WORK,), jnp.int32),
            pltpu.VMEM((WORK,), jnp.float32),
            pltpu.VMEM((WORK,), jnp.int32),
            pltpu.VMEM((NPAD,), jnp.int32),
            pltpu.VMEM((NPAD,), jnp.int32),
            pltpu.VMEM((NPAD,), jnp.int32),
            pltpu.VMEM((NPAD,), jnp.int32),
            pltpu.VMEM((NPAD,), jnp.int32),
            pltpu.VMEM((NPAD,), jnp.int32),
            pltpu.VMEM((NPAD,), jnp.int32),
        ],
    )
    def k(x_hbm, out_hbm, e_in, c1_in, ae, ap, be, bp, ce, cp,
          eb, mzl, mzh, myl, myh, mxl, mxh):
        wid = lax.axis_index("s") * 2 + lax.axis_index("c")

        @pl.when(wid < NSLICE)
        def _():
            pltpu.sync_copy(x_hbm.at[wid, 0], e_in)
            pltpu.sync_copy(x_hbm.at[wid, 1], c1_in)

            def init_body(j, cnt):
                ids = j * 16 + lax.iota(jnp.int32, 16)
                sl = pl.ds(j * 16, 16)
                wsl = pl.ds(OFF + j * 16, 16)
                e = e_in[sl]
                valid = jnp.logical_and(ids < N_CELL, e != 0.0)
                ae[wsl] = jnp.where(valid, e, -1.0)
                ap[wsl] = jnp.where(valid, ids * 2, BIGP)
                eb[sl] = jnp.where(
                    valid, lax.bitcast_convert_type(e, jnp.int32), -1)
                z = lax.rem(ids, 10)
                y = lax.rem(lax.div(ids, 10), 10)
                xx = lax.div(ids, 100)
                one = jnp.full((16,), 1, jnp.int32)
                zero = jnp.full((16,), 0, jnp.int32)
                mzl[sl] = jnp.where(z > 0, one, zero)
                mzh[sl] = jnp.where(z < 9, one, zero)
                myl[sl] = jnp.where(y > 0, one, zero)
                myh[sl] = jnp.where(y < 9, one, zero)
                mxl[sl] = jnp.where(xx > 0, one, zero)
                mxh[sl] = jnp.where(xx < 9, one, zero)
                return cnt + jnp.sum(jnp.where(valid, one, zero))

            und0 = lax.fori_loop(0, NCH, init_body, jnp.int32(0))

            def pool(src_e, src_p, dst_e, dst_p, stride, ml_ref, mh_ref):
                def body(j, carry):
                    b = OFF + j * 16
                    sl = pl.ds(j * 16, 16)
                    ce_ = src_e[pl.ds(b, 16)]
                    cp_ = src_p[pl.ds(b, 16)]
                    le = src_e[pl.ds(b - stride, 16)]
                    lp = src_p[pl.ds(b - stride, 16)]
                    re_ = src_e[pl.ds(b + stride, 16)]
                    rp = src_p[pl.ds(b + stride, 16)]
                    mlo = ml_ref[sl] != 0
                    mhi = mh_ref[sl] != 0
                    le = jnp.where(mlo, le, -2.0)
                    lp = jnp.where(mlo, lp, BIGP)
                    re_ = jnp.where(mhi, re_, -2.0)
                    rp = jnp.where(mhi, rp, BIGP)
                    gt = jnp.logical_or(
                        ce_ > le, jnp.logical_and(ce_ == le, cp_ < lp))
                    me = jnp.where(gt, ce_, le)
                    mp = jnp.where(gt, cp_, lp)
                    gt2 = jnp.logical_or(
                        me > re_, jnp.logical_and(me == re_, mp < rp))
                    dst_e[pl.ds(b, 16)] = jnp.where(gt2, me, re_)
                    dst_p[pl.ds(b, 16)] = jnp.where(gt2, mp, rp)
                    return carry
                lax.fori_loop(0, NCH, body, jnp.int32(0))

            def round_body(_c):
                pool(ae, ap, be, bp, 1, mzl, mzh)
                pool(be, bp, ce, cp, 10, myl, myh)
                pool(ce, cp, be, bp, 100, mxl, mxh)

                def upd(j, cnt):
                    b = pl.ds(OFF + j * 16, 16)
                    me = be[b]
                    mp = bp[b]
                    e = ae[b]
                    p = ap[b]
                    kb = (p & 1) == 1
                    und = jnp.logical_and(e > 0.0, jnp.logical_not(kb))
                    isself = jnp.logical_and(me == e, mp == p)
                    newk = jnp.logical_and(und, isself)
                    sup = und & ((mp & 1) == 1) & jnp.logical_not(isself)
                    ap[b] = jnp.where(newk, p | 1, p)
                    ae[b] = jnp.where(sup, -1.0, e)
                    rem = und & jnp.logical_not(newk) & jnp.logical_not(sup)
                    one = jnp.full((16,), 1, jnp.int32)
                    zero = jnp.full((16,), 0, jnp.int32)
                    return cnt + jnp.sum(jnp.where(rem, one, zero))

                return lax.fori_loop(0, NCH, upd, jnp.int32(0))

            lax.while_loop(lambda c: c > 0, round_body, und0)

            def kcount(j, cnt):
                p = ap[pl.ds(OFF + j * 16, 16)]
                return cnt + jnp.sum(p & 1)

            kept_n = lax.fori_loop(0, NCH, kcount, jnp.int32(0))

            @pl.when(kept_n > MAXEV)
            def _cap():
                one = jnp.full((16,), 1, jnp.int32)
                zero = jnp.full((16,), 0, jnp.int32)

                def count_gt(t):
                    def cb(j, cnt):
                        ebj = eb[pl.ds(j * 16, 16)]
                        return cnt + jnp.sum(jnp.where(ebj > t, one, zero))
                    return lax.fori_loop(0, NCH, cb, jnp.int32(0))

                def bs(_i, lohi):
                    lo, hi = lohi
                    mid = lax.div(lo + hi, jnp.int32(2))
                    pred = count_gt(mid) < MAXEV
                    return (jnp.where(pred, lo, mid + 1),
                            jnp.where(pred, mid, hi))

                tau, _ = lax.fori_loop(
                    0, 30, bs, (jnp.int32(0), jnp.int32((1 << 30) - 1)))
                quota = MAXEV - count_gt(tau)

                def capb(j, carry):
                    b = pl.ds(OFF + j * 16, 16)
                    sl = pl.ds(j * 16, 16)
                    ebj = eb[sl]
                    tie = ebj == tau
                    tc = jnp.where(tie, one, zero)
                    pfx = plsc.cumsum(tc)
                    surv = tie & ((carry + (pfx - tc)) < quota)
                    allow = (ebj > tau) | surv
                    p = ap[b]
                    ap[b] = jnp.where(allow, p, p & (~1))
                    return carry + jnp.max(pfx)

                lax.fori_loop(0, NCH, capb, jnp.int32(0))

            anyv = und0 > 0

            def ob(j, carry):
                b = pl.ds(OFF + j * 16, 16)
                sl = pl.ds(j * 16, 16)
                keepm = jnp.logical_or((ap[b] & 1) == 1,
                                       jnp.logical_not(anyv))
                e_in[sl] = jnp.where(keepm, e_in[sl], 0.0)
                c1_in[sl] = jnp.where(keepm, c1_in[sl], 0.0)
                return carry

            lax.fori_loop(0, NCH, ob, jnp.int32(0))
            pltpu.sync_copy(e_in, out_hbm.at[wid, 0])
            pltpu.sync_copy(c1_in, out_hbm.at[wid, 1])

    return k(xr)


def kernel(x):
    shape = x.shape
    xr = x.reshape(NSLICE, 2, N_CELL)
    xr = jnp.pad(xr, ((0, 0), (0, 0), (0, NPAD - N_CELL)))
    out = _nms_sc(xr)
    return out[:, :, :N_CELL].reshape(shape)


# speedup vs baseline: 53.9914x; 1.2461x over previous
import functools

import jax
import jax.numpy as jnp
from jax import lax
from jax.experimental import pallas as pl
from jax.experimental.pallas import tpu as pltpu
from jax.experimental.pallas import tpu_sc as plsc

N_CELL = 1000
NPAD = 1024
NCH = NPAD // 16
OFF = 112
WORK = OFF + NPAD + 112
BIGP = 1 << 29
MAXEV = 100
NSLICE = 16
UNROLL = 4


def _nms_sc(xr):
    mesh = plsc.VectorSubcoreMesh(core_axis_name="c", subcore_axis_name="s")

    @functools.partial(
        pl.kernel,
        out_type=jax.ShapeDtypeStruct((NSLICE, 2, NPAD), jnp.float32),
        mesh=mesh,
        compiler_params=pltpu.CompilerParams(needs_layout_passes=False),
        scratch_types=[
            pltpu.VMEM((NPAD,), jnp.float32),
            pltpu.VMEM((NPAD,), jnp.float32),
            pltpu.VMEM((WORK,), jnp.float32),
            pltpu.VMEM((WORK,), jnp.int32),
            pltpu.VMEM((WORK,), jnp.float32),
            pltpu.VMEM((WORK,), jnp.int32),
            pltpu.VMEM((WORK,), jnp.float32),
            pltpu.VMEM((WORK,), jnp.int32),
            pltpu.VMEM((NPAD,), jnp.int32),
            pltpu.VMEM((NPAD,), jnp.int32),
            pltpu.VMEM((NPAD,), jnp.int32),
            pltpu.VMEM((NPAD,), jnp.int32),
        ],
    )
    def k(x_hbm, out_hbm, e_in, c1_in, ae, ap, be, bp, ce, cp,
          eb, mz, my, mx):
        wid = lax.axis_index("s") * 2 + lax.axis_index("c")

        @pl.when(wid < NSLICE)
        def _():
            pltpu.sync_copy(x_hbm.at[wid, 0], e_in)
            pltpu.sync_copy(x_hbm.at[wid, 1], c1_in)

            zerov = jnp.full((16,), 0, jnp.int32)

            @plsc.parallel_loop(0, NCH, unroll=UNROLL, carry=zerov)
            def und0_vec(j, cnt):
                ids = j * 16 + lax.iota(jnp.int32, 16)
                sl = pl.ds(j * 16, 16)
                wsl = pl.ds(OFF + j * 16, 16)
                e = e_in[sl]
                valid = jnp.logical_and(ids < N_CELL, e != 0.0)
                ae[wsl] = jnp.where(valid, e, -1.0)
                ap[wsl] = jnp.where(valid, ids * 2, BIGP)
                eb[sl] = jnp.where(
                    valid, lax.bitcast_convert_type(e, jnp.int32), -1)
                z = lax.rem(ids, 10)
                y = lax.rem(lax.div(ids, 10), 10)
                xx = lax.div(ids, 100)
                one = jnp.full((16,), 1, jnp.int32)
                two = jnp.full((16,), 2, jnp.int32)
                mz[sl] = jnp.where(z > 0, one, zerov) | jnp.where(
                    z < 9, two, zerov)
                my[sl] = jnp.where(y > 0, one, zerov) | jnp.where(
                    y < 9, two, zerov)
                mx[sl] = jnp.where(xx > 0, one, zerov) | jnp.where(
                    xx < 9, two, zerov)
                return cnt + valid.astype(jnp.int32)

            und0 = jnp.sum(und0_vec)

            def pool(src_e, src_p, dst_e, dst_p, stride, m_ref):
                @plsc.parallel_loop(0, NCH, unroll=UNROLL)
                def _body(j):
                    b = OFF + j * 16
                    m = m_ref[pl.ds(j * 16, 16)]
                    ce_ = src_e[pl.ds(b, 16)]
                    cp_ = src_p[pl.ds(b, 16)]
                    le = src_e[pl.ds(b - stride, 16)]
                    lp = src_p[pl.ds(b - stride, 16)]
                    re_ = src_e[pl.ds(b + stride, 16)]
                    rp = src_p[pl.ds(b + stride, 16)]
                    mlo = (m & 1) != 0
                    mhi = (m & 2) != 0
                    le = jnp.where(mlo, le, -2.0)
                    lp = jnp.where(mlo, lp, BIGP)
                    re_ = jnp.where(mhi, re_, -2.0)
                    rp = jnp.where(mhi, rp, BIGP)
                    gt = jnp.logical_or(
                        ce_ > le, jnp.logical_and(ce_ == le, cp_ < lp))
                    me = jnp.where(gt, ce_, le)
                    mp = jnp.where(gt, cp_, lp)
                    gt2 = jnp.logical_or(
                        me > re_, jnp.logical_and(me == re_, mp < rp))
                    dst_e[pl.ds(b, 16)] = jnp.where(gt2, me, re_)
                    dst_p[pl.ds(b, 16)] = jnp.where(gt2, mp, rp)

            def round_body(_c):
                pool(ae, ap, be, bp, 1, mz)
                pool(be, bp, ce, cp, 10, my)

                @plsc.parallel_loop(0, NCH, unroll=UNROLL, carry=zerov)
                def rem_vec(j, cnt):
                    b = OFF + j * 16
                    sl = pl.ds(b, 16)
                    m = mx[pl.ds(j * 16, 16)]
                    ce_ = ce[sl]
                    cp_ = cp[sl]
                    le = ce[pl.ds(b - 100, 16)]
                    lp = cp[pl.ds(b - 100, 16)]
                    re_ = ce[pl.ds(b + 100, 16)]
                    rp = cp[pl.ds(b + 100, 16)]
                    mlo = (m & 1) != 0
                    mhi = (m & 2) != 0
                    le = jnp.where(mlo, le, -2.0)
                    lp = jnp.where(mlo, lp, BIGP)
                    re_ = jnp.where(mhi, re_, -2.0)
                    rp = jnp.where(mhi, rp, BIGP)
                    gt = jnp.logical_or(
                        ce_ > le, jnp.logical_and(ce_ == le, cp_ < lp))
                    me = jnp.where(gt, ce_, le)
                    mp = jnp.where(gt, cp_, lp)
                    gt2 = jnp.logical_or(
                        me > re_, jnp.logical_and(me == re_, mp < rp))
                    me = jnp.where(gt2, me, re_)
                    mp = jnp.where(gt2, mp, rp)
                    e = ae[sl]
                    p = ap[sl]
                    kb = (p & 1) == 1
                    und = jnp.logical_and(e > 0.0, jnp.logical_not(kb))
                    isself = jnp.logical_and(me == e, mp == p)
                    newk = jnp.logical_and(und, isself)
                    sup = und & ((mp & 1) == 1) & jnp.logical_not(isself)
                    ap[sl] = jnp.where(newk, p | 1, p)
                    ae[sl] = jnp.where(sup, -1.0, e)
                    rem = und & jnp.logical_not(newk) & jnp.logical_not(sup)
                    return cnt + rem.astype(jnp.int32)

                return jnp.sum(rem_vec)

            lax.while_loop(lambda c: c > 0, round_body, und0)

            @plsc.parallel_loop(0, NCH, unroll=UNROLL, carry=zerov)
            def kept_vec(j, cnt):
                p = ap[pl.ds(OFF + j * 16, 16)]
                return cnt + (p & 1)

            kept_n = jnp.sum(kept_vec)

            @pl.when(kept_n > MAXEV)
            def _cap():
                def count_gt(t):
                    @plsc.parallel_loop(0, NCH, unroll=UNROLL, carry=zerov)
                    def cvec(j, cnt):
                        ebj = eb[pl.ds(j * 16, 16)]
                        return cnt + (ebj > t).astype(jnp.int32)
                    return jnp.sum(cvec)

                def bs(_i, lohi):
                    lo, hi = lohi
                    mid = lax.div(lo + hi, jnp.int32(2))
                    pred = count_gt(mid) < MAXEV
                    return (jnp.where(pred, lo, mid + 1),
                            jnp.where(pred, mid, hi))

                tau, _ = lax.fori_loop(
                    0, 30, bs, (jnp.int32(0), jnp.int32((1 << 30) - 1)))
                quota = MAXEV - count_gt(tau)

                def capb(j, carry):
                    b = pl.ds(OFF + j * 16, 16)
                    sl = pl.ds(j * 16, 16)
                    ebj = eb[sl]
                    tie = ebj == tau
                    tc = tie.astype(jnp.int32)
                    pfx = plsc.cumsum(tc)
                    surv = tie & ((carry + (pfx - tc)) < quota)
                    allow = (ebj > tau) | surv
                    p = ap[b]
                    ap[b] = jnp.where(allow, p, p & (~1))
                    return carry + jnp.max(pfx)

                lax.fori_loop(0, NCH, capb, jnp.int32(0))

            anyv = und0 > 0

            @plsc.parallel_loop(0, NCH, unroll=UNROLL)
            def _ob(j):
                b = pl.ds(OFF + j * 16, 16)
                sl = pl.ds(j * 16, 16)
                keepm = jnp.logical_or((ap[b] & 1) == 1,
                                       jnp.logical_not(anyv))
                e_in[sl] = jnp.where(keepm, e_in[sl], 0.0)
                c1_in[sl] = jnp.where(keepm, c1_in[sl], 0.0)

            pltpu.sync_copy(e_in, out_hbm.at[wid, 0])
            pltpu.sync_copy(c1_in, out_hbm.at[wid, 1])

    return k(xr)


def kernel(x):
    shape = x.shape
    xr = x.reshape(NSLICE, 2, N_CELL)
    xr = jnp.pad(xr, ((0, 0), (0, 0), (0, NPAD - N_CELL)))
    out = _nms_sc(xr)
    return out[:, :, :N_CELL].reshape(shape)


# speedup vs baseline: 55.8275x; 1.0340x over previous
import functools

import jax
import jax.numpy as jnp
from jax import lax
from jax.experimental import pallas as pl
from jax.experimental.pallas import tpu as pltpu
from jax.experimental.pallas import tpu_sc as plsc

N_CELL = 1000
NPAD = 1024
NCH = NPAD // 16
OFF = 112
WORK = OFF + NPAD + 112
BIGP = 1 << 29
MAXEV = 100
NSLICE = 16
UNROLL = 4


def _nms_sc(xr):
    mesh = plsc.VectorSubcoreMesh(
        core_axis_name="c", subcore_axis_name="s", num_cores=1)

    @functools.partial(
        pl.kernel,
        out_type=jax.ShapeDtypeStruct((NSLICE, 2, NPAD), jnp.float32),
        mesh=mesh,
        compiler_params=pltpu.CompilerParams(needs_layout_passes=False),
        scratch_types=[
            pltpu.VMEM((NPAD,), jnp.float32),
            pltpu.VMEM((NPAD,), jnp.float32),
            pltpu.VMEM((WORK,), jnp.float32),
            pltpu.VMEM((WORK,), jnp.int32),
            pltpu.VMEM((WORK,), jnp.float32),
            pltpu.VMEM((WORK,), jnp.int32),
            pltpu.VMEM((WORK,), jnp.float32),
            pltpu.VMEM((WORK,), jnp.int32),
            pltpu.VMEM((NPAD,), jnp.int32),
            pltpu.VMEM((NPAD,), jnp.int32),
            pltpu.VMEM((NPAD,), jnp.int32),
            pltpu.VMEM((NPAD,), jnp.int32),
        ],
    )
    def k(x_hbm, out_hbm, e_in, c1_in, ae, ap, be, bp, ce, cp,
          eb, mz, my, mx):
        wid = lax.axis_index("s")

        @pl.when(wid < NSLICE)
        def _():
            pltpu.sync_copy(x_hbm.at[wid, 0], e_in)
            pltpu.sync_copy(x_hbm.at[wid, 1], c1_in)

            zerov = jnp.full((16,), 0, jnp.int32)

            @plsc.parallel_loop(0, NCH, unroll=UNROLL, carry=zerov)
            def und0_vec(j, cnt):
                ids = j * 16 + lax.iota(jnp.int32, 16)
                sl = pl.ds(j * 16, 16)
                wsl = pl.ds(OFF + j * 16, 16)
                e = e_in[sl]
                valid = jnp.logical_and(ids < N_CELL, e != 0.0)
                ae[wsl] = jnp.where(valid, e, -1.0)
                ap[wsl] = jnp.where(valid, ids * 2, BIGP)
                eb[sl] = jnp.where(
                    valid, lax.bitcast_convert_type(e, jnp.int32), -1)
                z = lax.rem(ids, 10)
                y = lax.rem(lax.div(ids, 10), 10)
                xx = lax.div(ids, 100)
                one = jnp.full((16,), 1, jnp.int32)
                two = jnp.full((16,), 2, jnp.int32)
                mz[sl] = jnp.where(z > 0, one, zerov) | jnp.where(
                    z < 9, two, zerov)
                my[sl] = jnp.where(y > 0, one, zerov) | jnp.where(
                    y < 9, two, zerov)
                mx[sl] = jnp.where(xx > 0, one, zerov) | jnp.where(
                    xx < 9, two, zerov)
                return cnt + valid.astype(jnp.int32)

            und0 = jnp.sum(und0_vec)

            def pool(src_e, src_p, dst_e, dst_p, stride, m_ref):
                @plsc.parallel_loop(0, NCH, unroll=UNROLL)
                def _body(j):
                    b = OFF + j * 16
                    m = m_ref[pl.ds(j * 16, 16)]
                    ce_ = src_e[pl.ds(b, 16)]
                    cp_ = src_p[pl.ds(b, 16)]
                    le = src_e[pl.ds(b - stride, 16)]
                    lp = src_p[pl.ds(b - stride, 16)]
                    re_ = src_e[pl.ds(b + stride, 16)]
                    rp = src_p[pl.ds(b + stride, 16)]
                    mlo = (m & 1) != 0
                    mhi = (m & 2) != 0
                    le = jnp.where(mlo, le, -2.0)
                    lp = jnp.where(mlo, lp, BIGP)
                    re_ = jnp.where(mhi, re_, -2.0)
                    rp = jnp.where(mhi, rp, BIGP)
                    gt = jnp.logical_or(
                        ce_ > le, jnp.logical_and(ce_ == le, cp_ < lp))
                    me = jnp.where(gt, ce_, le)
                    mp = jnp.where(gt, cp_, lp)
                    gt2 = jnp.logical_or(
                        me > re_, jnp.logical_and(me == re_, mp < rp))
                    dst_e[pl.ds(b, 16)] = jnp.where(gt2, me, re_)
                    dst_p[pl.ds(b, 16)] = jnp.where(gt2, mp, rp)

            def round_body(_c):
                pool(ae, ap, be, bp, 1, mz)
                pool(be, bp, ce, cp, 10, my)

                @plsc.parallel_loop(0, NCH, unroll=UNROLL, carry=zerov)
                def rem_vec(j, cnt):
                    b = OFF + j * 16
                    sl = pl.ds(b, 16)
                    m = mx[pl.ds(j * 16, 16)]
                    ce_ = ce[sl]
                    cp_ = cp[sl]
                    le = ce[pl.ds(b - 100, 16)]
                    lp = cp[pl.ds(b - 100, 16)]
                    re_ = ce[pl.ds(b + 100, 16)]
                    rp = cp[pl.ds(b + 100, 16)]
                    mlo = (m & 1) != 0
                    mhi = (m & 2) != 0
                    le = jnp.where(mlo, le, -2.0)
                    lp = jnp.where(mlo, lp, BIGP)
                    re_ = jnp.where(mhi, re_, -2.0)
                    rp = jnp.where(mhi, rp, BIGP)
                    gt = jnp.logical_or(
                        ce_ > le, jnp.logical_and(ce_ == le, cp_ < lp))
                    me = jnp.where(gt, ce_, le)
                    mp = jnp.where(gt, cp_, lp)
                    gt2 = jnp.logical_or(
                        me > re_, jnp.logical_and(me == re_, mp < rp))
                    me = jnp.where(gt2, me, re_)
                    mp = jnp.where(gt2, mp, rp)
                    e = ae[sl]
                    p = ap[sl]
                    kb = (p & 1) == 1
                    und = jnp.logical_and(e > 0.0, jnp.logical_not(kb))
                    isself = jnp.logical_and(me == e, mp == p)
                    newk = jnp.logical_and(und, isself)
                    sup = und & ((mp & 1) == 1) & jnp.logical_not(isself)
                    ap[sl] = jnp.where(newk, p | 1, p)
                    ae[sl] = jnp.where(sup, -1.0, e)
                    rem = und & jnp.logical_not(newk) & jnp.logical_not(sup)
                    return cnt + rem.astype(jnp.int32)

                return jnp.sum(rem_vec)

            lax.while_loop(lambda c: c > 0, round_body, und0)

            @plsc.parallel_loop(0, NCH, unroll=UNROLL, carry=zerov)
            def kept_vec(j, cnt):
                p = ap[pl.ds(OFF + j * 16, 16)]
                return cnt + (p & 1)

            kept_n = jnp.sum(kept_vec)

            @pl.when(kept_n > MAXEV)
            def _cap():
                def count_gt(t):
                    @plsc.parallel_loop(0, NCH, unroll=UNROLL, carry=zerov)
                    def cvec(j, cnt):
                        ebj = eb[pl.ds(j * 16, 16)]
                        return cnt + (ebj > t).astype(jnp.int32)
                    return jnp.sum(cvec)

                def bs(_i, lohi):
                    lo, hi = lohi
                    mid = lax.div(lo + hi, jnp.int32(2))
                    pred = count_gt(mid) < MAXEV
                    return (jnp.where(pred, lo, mid + 1),
                            jnp.where(pred, mid, hi))

                tau, _ = lax.fori_loop(
                    0, 30, bs, (jnp.int32(0), jnp.int32((1 << 30) - 1)))
                quota = MAXEV - count_gt(tau)

                def capb(j, carry):
                    b = pl.ds(OFF + j * 16, 16)
                    sl = pl.ds(j * 16, 16)
                    ebj = eb[sl]
                    tie = ebj == tau
                    tc = tie.astype(jnp.int32)
                    pfx = plsc.cumsum(tc)
                    surv = tie & ((carry + (pfx - tc)) < quota)
                    allow = (ebj > tau) | surv
                    p = ap[b]
                    ap[b] = jnp.where(allow, p, p & (~1))
                    return carry + jnp.max(pfx)

                lax.fori_loop(0, NCH, capb, jnp.int32(0))

            anyv = und0 > 0

            @plsc.parallel_loop(0, NCH, unroll=UNROLL)
            def _ob(j):
                b = pl.ds(OFF + j * 16, 16)
                sl = pl.ds(j * 16, 16)
                keepm = jnp.logical_or((ap[b] & 1) == 1,
                                       jnp.logical_not(anyv))
                e_in[sl] = jnp.where(keepm, e_in[sl], 0.0)
                c1_in[sl] = jnp.where(keepm, c1_in[sl], 0.0)

            pltpu.sync_copy(e_in, out_hbm.at[wid, 0])
            pltpu.sync_copy(c1_in, out_hbm.at[wid, 1])

    return k(xr)


def kernel(x):
    shape = x.shape
    xr = x.reshape(NSLICE, 2, N_CELL)
    xr = jnp.pad(xr, ((0, 0), (0, 0), (0, NPAD - N_CELL)))
    out = _nms_sc(xr)
    return out[:, :, :N_CELL].reshape(shape)


# speedup vs baseline: 56.6604x; 1.0149x over previous
import functools

import jax
import jax.numpy as jnp
from jax import lax
from jax.experimental import pallas as pl
from jax.experimental.pallas import tpu as pltpu
from jax.experimental.pallas import tpu_sc as plsc

N_CELL = 1000
NPAD = 1024
NCH = NPAD // 16
OFF = 112
WORK = OFF + NPAD + 112
BIGP = 1 << 29
MAXEV = 100
NSLICE = 16
UNROLL = 8


def _nms_sc(xr):
    mesh = plsc.VectorSubcoreMesh(
        core_axis_name="c", subcore_axis_name="s", num_cores=1)

    @functools.partial(
        pl.kernel,
        out_type=jax.ShapeDtypeStruct((NSLICE, 2, NPAD), jnp.float32),
        mesh=mesh,
        compiler_params=pltpu.CompilerParams(needs_layout_passes=False),
        scratch_types=[
            pltpu.VMEM((NPAD,), jnp.float32),
            pltpu.VMEM((NPAD,), jnp.float32),
            pltpu.VMEM((WORK,), jnp.float32),
            pltpu.VMEM((WORK,), jnp.int32),
            pltpu.VMEM((WORK,), jnp.float32),
            pltpu.VMEM((WORK,), jnp.int32),
            pltpu.VMEM((WORK,), jnp.float32),
            pltpu.VMEM((WORK,), jnp.int32),
            pltpu.VMEM((NPAD,), jnp.int32),
            pltpu.VMEM((NPAD,), jnp.int32),
            pltpu.VMEM((NPAD,), jnp.int32),
            pltpu.VMEM((NPAD,), jnp.int32),
        ],
    )
    def k(x_hbm, out_hbm, e_in, c1_in, ae, ap, be, bp, ce, cp,
          eb, mz, my, mx):
        wid = lax.axis_index("s")

        @pl.when(wid < NSLICE)
        def _():
            pltpu.sync_copy(x_hbm.at[wid, 0], e_in)
            pltpu.sync_copy(x_hbm.at[wid, 1], c1_in)

            zerov = jnp.full((16,), 0, jnp.int32)

            @plsc.parallel_loop(0, NCH, unroll=UNROLL, carry=zerov)
            def und0_vec(j, cnt):
                ids = j * 16 + lax.iota(jnp.int32, 16)
                sl = pl.ds(j * 16, 16)
                wsl = pl.ds(OFF + j * 16, 16)
                e = e_in[sl]
                valid = jnp.logical_and(ids < N_CELL, e != 0.0)
                ae[wsl] = jnp.where(valid, e, -1.0)
                ap[wsl] = jnp.where(valid, ids * 2, BIGP)
                eb[sl] = jnp.where(
                    valid, lax.bitcast_convert_type(e, jnp.int32), -1)
                z = lax.rem(ids, 10)
                y = lax.rem(lax.div(ids, 10), 10)
                xx = lax.div(ids, 100)
                one = jnp.full((16,), 1, jnp.int32)
                two = jnp.full((16,), 2, jnp.int32)
                mz[sl] = jnp.where(z > 0, one, zerov) | jnp.where(
                    z < 9, two, zerov)
                my[sl] = jnp.where(y > 0, one, zerov) | jnp.where(
                    y < 9, two, zerov)
                mx[sl] = jnp.where(xx > 0, one, zerov) | jnp.where(
                    xx < 9, two, zerov)
                return cnt + valid.astype(jnp.int32)

            und0 = jnp.sum(und0_vec)

            def pool(src_e, src_p, dst_e, dst_p, stride, m_ref):
                @plsc.parallel_loop(0, NCH, unroll=UNROLL)
                def _body(j):
                    b = OFF + j * 16
                    m = m_ref[pl.ds(j * 16, 16)]
                    ce_ = src_e[pl.ds(b, 16)]
                    cp_ = src_p[pl.ds(b, 16)]
                    le = src_e[pl.ds(b - stride, 16)]
                    lp = src_p[pl.ds(b - stride, 16)]
                    re_ = src_e[pl.ds(b + stride, 16)]
                    rp = src_p[pl.ds(b + stride, 16)]
                    mlo = (m & 1) != 0
                    mhi = (m & 2) != 0
                    le = jnp.where(mlo, le, -2.0)
                    lp = jnp.where(mlo, lp, BIGP)
                    re_ = jnp.where(mhi, re_, -2.0)
                    rp = jnp.where(mhi, rp, BIGP)
                    gt = jnp.logical_or(
                        ce_ > le, jnp.logical_and(ce_ == le, cp_ < lp))
                    me = jnp.where(gt, ce_, le)
                    mp = jnp.where(gt, cp_, lp)
                    gt2 = jnp.logical_or(
                        me > re_, jnp.logical_and(me == re_, mp < rp))
                    dst_e[pl.ds(b, 16)] = jnp.where(gt2, me, re_)
                    dst_p[pl.ds(b, 16)] = jnp.where(gt2, mp, rp)

            def round_body(_c):
                pool(ae, ap, be, bp, 1, mz)
                pool(be, bp, ce, cp, 10, my)

                @plsc.parallel_loop(0, NCH, unroll=UNROLL, carry=zerov)
                def rem_vec(j, cnt):
                    b = OFF + j * 16
                    sl = pl.ds(b, 16)
                    m = mx[pl.ds(j * 16, 16)]
                    ce_ = ce[sl]
                    cp_ = cp[sl]
                    le = ce[pl.ds(b - 100, 16)]
                    lp = cp[pl.ds(b - 100, 16)]
                    re_ = ce[pl.ds(b + 100, 16)]
                    rp = cp[pl.ds(b + 100, 16)]
                    mlo = (m & 1) != 0
                    mhi = (m & 2) != 0
                    le = jnp.where(mlo, le, -2.0)
                    lp = jnp.where(mlo, lp, BIGP)
                    re_ = jnp.where(mhi, re_, -2.0)
                    rp = jnp.where(mhi, rp, BIGP)
                    gt = jnp.logical_or(
                        ce_ > le, jnp.logical_and(ce_ == le, cp_ < lp))
                    me = jnp.where(gt, ce_, le)
                    mp = jnp.where(gt, cp_, lp)
                    gt2 = jnp.logical_or(
                        me > re_, jnp.logical_and(me == re_, mp < rp))
                    me = jnp.where(gt2, me, re_)
                    mp = jnp.where(gt2, mp, rp)
                    e = ae[sl]
                    p = ap[sl]
                    kb = (p & 1) == 1
                    und = jnp.logical_and(e > 0.0, jnp.logical_not(kb))
                    isself = jnp.logical_and(me == e, mp == p)
                    newk = jnp.logical_and(und, isself)
                    sup = und & ((mp & 1) == 1) & jnp.logical_not(isself)
                    ap[sl] = jnp.where(newk, p | 1, p)
                    ae[sl] = jnp.where(sup, -1.0, e)
                    rem = und & jnp.logical_not(newk) & jnp.logical_not(sup)
                    return cnt + rem.astype(jnp.int32)

                return jnp.sum(rem_vec)

            lax.while_loop(lambda c: c > 0, round_body, und0)

            @plsc.parallel_loop(0, NCH, unroll=UNROLL, carry=zerov)
            def kept_vec(j, cnt):
                p = ap[pl.ds(OFF + j * 16, 16)]
                return cnt + (p & 1)

            kept_n = jnp.sum(kept_vec)

            @pl.when(kept_n > MAXEV)
            def _cap():
                def count_gt(t):
                    @plsc.parallel_loop(0, NCH, unroll=UNROLL, carry=zerov)
                    def cvec(j, cnt):
                        ebj = eb[pl.ds(j * 16, 16)]
                        return cnt + (ebj > t).astype(jnp.int32)
                    return jnp.sum(cvec)

                def bs(_i, lohi):
                    lo, hi = lohi
                    mid = lax.div(lo + hi, jnp.int32(2))
                    pred = count_gt(mid) < MAXEV
                    return (jnp.where(pred, lo, mid + 1),
                            jnp.where(pred, mid, hi))

                tau, _ = lax.fori_loop(
                    0, 30, bs, (jnp.int32(0), jnp.int32((1 << 30) - 1)))
                quota = MAXEV - count_gt(tau)

                def capb(j, carry):
                    b = pl.ds(OFF + j * 16, 16)
                    sl = pl.ds(j * 16, 16)
                    ebj = eb[sl]
                    tie = ebj == tau
                    tc = tie.astype(jnp.int32)
                    pfx = plsc.cumsum(tc)
                    surv = tie & ((carry + (pfx - tc)) < quota)
                    allow = (ebj > tau) | surv
                    p = ap[b]
                    ap[b] = jnp.where(allow, p, p & (~1))
                    return carry + jnp.max(pfx)

                lax.fori_loop(0, NCH, capb, jnp.int32(0))

            anyv = und0 > 0

            @plsc.parallel_loop(0, NCH, unroll=UNROLL)
            def _ob(j):
                b = pl.ds(OFF + j * 16, 16)
                sl = pl.ds(j * 16, 16)
                keepm = jnp.logical_or((ap[b] & 1) == 1,
                                       jnp.logical_not(anyv))
                e_in[sl] = jnp.where(keepm, e_in[sl], 0.0)
                c1_in[sl] = jnp.where(keepm, c1_in[sl], 0.0)

            pltpu.sync_copy(e_in, out_hbm.at[wid, 0])
            pltpu.sync_copy(c1_in, out_hbm.at[wid, 1])

    return k(xr)


def kernel(x):
    shape = x.shape
    xr = x.reshape(NSLICE, 2, N_CELL)
    xr = jnp.pad(xr, ((0, 0), (0, 0), (0, NPAD - N_CELL)))
    out = _nms_sc(xr)
    return out[:, :, :N_CELL].reshape(shape)


# speedup vs baseline: 58.0433x; 1.0244x over previous
import functools

import jax
import jax.numpy as jnp
from jax import lax
from jax.experimental import pallas as pl
from jax.experimental.pallas import tpu as pltpu
from jax.experimental.pallas import tpu_sc as plsc

N_CELL = 1000
NPAD = 1024
NCH = NPAD // 16
OFF = 112
WORK = OFF + NPAD + 112
BIGP = 1 << 29
MAXEV = 100
NSLICE = 16
UNROLL = 8


def _nms_sc(xr):
    mesh = plsc.VectorSubcoreMesh(
        core_axis_name="c", subcore_axis_name="s", num_cores=1)

    @functools.partial(
        pl.kernel,
        out_type=jax.ShapeDtypeStruct((NSLICE * 2 * N_CELL,), jnp.float32),
        mesh=mesh,
        compiler_params=pltpu.CompilerParams(needs_layout_passes=False),
        scratch_types=[
            pltpu.VMEM((NPAD,), jnp.float32),
            pltpu.VMEM((NPAD,), jnp.float32),
            pltpu.VMEM((WORK,), jnp.float32),
            pltpu.VMEM((WORK,), jnp.int32),
            pltpu.VMEM((WORK,), jnp.float32),
            pltpu.VMEM((WORK,), jnp.int32),
            pltpu.VMEM((WORK,), jnp.float32),
            pltpu.VMEM((WORK,), jnp.int32),
            pltpu.VMEM((NPAD,), jnp.int32),
            pltpu.VMEM((NPAD,), jnp.int32),
            pltpu.VMEM((NPAD,), jnp.int32),
            pltpu.VMEM((NPAD,), jnp.int32),
            pltpu.SemaphoreType.DMA,
        ],
    )
    def k(x_hbm, out_hbm, e_in, c1_in, ae, ap, be, bp, ce, cp,
          eb, mz, my, mx, c1_sem):
        wid = lax.axis_index("s")

        @pl.when(wid < NSLICE)
        def _():
            base = wid * (2 * N_CELL)
            c1_cp = pltpu.make_async_copy(
                x_hbm.at[pl.ds(base + N_CELL, N_CELL)],
                c1_in.at[pl.ds(0, N_CELL)], c1_sem)
            c1_cp.start()
            pltpu.sync_copy(x_hbm.at[pl.ds(base, N_CELL)],
                            e_in.at[pl.ds(0, N_CELL)])

            zerov = jnp.full((16,), 0, jnp.int32)

            @plsc.parallel_loop(0, NCH, unroll=UNROLL, carry=zerov)
            def und0_vec(j, cnt):
                ids = j * 16 + lax.iota(jnp.int32, 16)
                sl = pl.ds(j * 16, 16)
                wsl = pl.ds(OFF + j * 16, 16)
                e = e_in[sl]
                valid = jnp.logical_and(ids < N_CELL, e != 0.0)
                ae[wsl] = jnp.where(valid, e, -1.0)
                ap[wsl] = jnp.where(valid, ids * 2, BIGP)
                eb[sl] = jnp.where(
                    valid, lax.bitcast_convert_type(e, jnp.int32), -1)
                z = lax.rem(ids, 10)
                y = lax.rem(lax.div(ids, 10), 10)
                xx = lax.div(ids, 100)
                one = jnp.full((16,), 1, jnp.int32)
                two = jnp.full((16,), 2, jnp.int32)
                mz[sl] = jnp.where(z > 0, one, zerov) | jnp.where(
                    z < 9, two, zerov)
                my[sl] = jnp.where(y > 0, one, zerov) | jnp.where(
                    y < 9, two, zerov)
                mx[sl] = jnp.where(xx > 0, one, zerov) | jnp.where(
                    xx < 9, two, zerov)
                return cnt + valid.astype(jnp.int32)

            und0 = jnp.sum(und0_vec)

            def pool(src_e, src_p, dst_e, dst_p, stride, m_ref):
                @plsc.parallel_loop(0, NCH, unroll=UNROLL)
                def _body(j):
                    b = OFF + j * 16
                    m = m_ref[pl.ds(j * 16, 16)]
                    ce_ = src_e[pl.ds(b, 16)]
                    cp_ = src_p[pl.ds(b, 16)]
                    le = src_e[pl.ds(b - stride, 16)]
                    lp = src_p[pl.ds(b - stride, 16)]
                    re_ = src_e[pl.ds(b + stride, 16)]
                    rp = src_p[pl.ds(b + stride, 16)]
                    mlo = (m & 1) != 0
                    mhi = (m & 2) != 0
                    le = jnp.where(mlo, le, -2.0)
                    lp = jnp.where(mlo, lp, BIGP)
                    re_ = jnp.where(mhi, re_, -2.0)
                    rp = jnp.where(mhi, rp, BIGP)
                    gt = jnp.logical_or(
                        ce_ > le, jnp.logical_and(ce_ == le, cp_ < lp))
                    me = jnp.where(gt, ce_, le)
                    mp = jnp.where(gt, cp_, lp)
                    gt2 = jnp.logical_or(
                        me > re_, jnp.logical_and(me == re_, mp < rp))
                    dst_e[pl.ds(b, 16)] = jnp.where(gt2, me, re_)
                    dst_p[pl.ds(b, 16)] = jnp.where(gt2, mp, rp)

            def round_body(_c):
                pool(ae, ap, be, bp, 1, mz)
                pool(be, bp, ce, cp, 10, my)

                @plsc.parallel_loop(0, NCH, unroll=UNROLL, carry=zerov)
                def rem_vec(j, cnt):
                    b = OFF + j * 16
                    sl = pl.ds(b, 16)
                    m = mx[pl.ds(j * 16, 16)]
                    ce_ = ce[sl]
                    cp_ = cp[sl]
                    le = ce[pl.ds(b - 100, 16)]
                    lp = cp[pl.ds(b - 100, 16)]
                    re_ = ce[pl.ds(b + 100, 16)]
                    rp = cp[pl.ds(b + 100, 16)]
                    mlo = (m & 1) != 0
                    mhi = (m & 2) != 0
                    le = jnp.where(mlo, le, -2.0)
                    lp = jnp.where(mlo, lp, BIGP)
                    re_ = jnp.where(mhi, re_, -2.0)
                    rp = jnp.where(mhi, rp, BIGP)
                    gt = jnp.logical_or(
                        ce_ > le, jnp.logical_and(ce_ == le, cp_ < lp))
                    me = jnp.where(gt, ce_, le)
                    mp = jnp.where(gt, cp_, lp)
                    gt2 = jnp.logical_or(
                        me > re_, jnp.logical_and(me == re_, mp < rp))
                    me = jnp.where(gt2, me, re_)
                    mp = jnp.where(gt2, mp, rp)
                    e = ae[sl]
                    p = ap[sl]
                    kb = (p & 1) == 1
                    und = jnp.logical_and(e > 0.0, jnp.logical_not(kb))
                    isself = jnp.logical_and(me == e, mp == p)
                    newk = jnp.logical_and(und, isself)
                    sup = und & ((mp & 1) == 1) & jnp.logical_not(isself)
                    ap[sl] = jnp.where(newk, p | 1, p)
                    ae[sl] = jnp.where(sup, -1.0, e)
                    rem = und & jnp.logical_not(newk) & jnp.logical_not(sup)
                    return cnt + rem.astype(jnp.int32)

                return jnp.sum(rem_vec)

            lax.while_loop(lambda c: c > 0, round_body, und0)

            @plsc.parallel_loop(0, NCH, unroll=UNROLL, carry=zerov)
            def kept_vec(j, cnt):
                p = ap[pl.ds(OFF + j * 16, 16)]
                return cnt + (p & 1)

            kept_n = jnp.sum(kept_vec)

            @pl.when(kept_n > MAXEV)
            def _cap():
                def count_gt(t):
                    @plsc.parallel_loop(0, NCH, unroll=UNROLL, carry=zerov)
                    def cvec(j, cnt):
                        ebj = eb[pl.ds(j * 16, 16)]
                        return cnt + (ebj > t).astype(jnp.int32)
                    return jnp.sum(cvec)

                def bs(_i, lohi):
                    lo, hi = lohi
                    mid = lax.div(lo + hi, jnp.int32(2))
                    pred = count_gt(mid) < MAXEV
                    return (jnp.where(pred, lo, mid + 1),
                            jnp.where(pred, mid, hi))

                tau, _ = lax.fori_loop(
                    0, 30, bs, (jnp.int32(0), jnp.int32((1 << 30) - 1)))
                quota = MAXEV - count_gt(tau)

                def capb(j, carry):
                    b = pl.ds(OFF + j * 16, 16)
                    sl = pl.ds(j * 16, 16)
                    ebj = eb[sl]
                    tie = ebj == tau
                    tc = tie.astype(jnp.int32)
                    pfx = plsc.cumsum(tc)
                    surv = tie & ((carry + (pfx - tc)) < quota)
                    allow = (ebj > tau) | surv
                    p = ap[b]
                    ap[b] = jnp.where(allow, p, p & (~1))
                    return carry + jnp.max(pfx)

                lax.fori_loop(0, NCH, capb, jnp.int32(0))

            anyv = und0 > 0
            c1_cp.wait()

            @plsc.parallel_loop(0, NCH, unroll=UNROLL)
            def _ob(j):
                b = pl.ds(OFF + j * 16, 16)
                sl = pl.ds(j * 16, 16)
                keepm = jnp.logical_or((ap[b] & 1) == 1,
                                       jnp.logical_not(anyv))
                e_in[sl] = jnp.where(keepm, e_in[sl], 0.0)
                c1_in[sl] = jnp.where(keepm, c1_in[sl], 0.0)

            pltpu.sync_copy(e_in.at[pl.ds(0, N_CELL)],
                            out_hbm.at[pl.ds(base, N_CELL)])
            pltpu.sync_copy(c1_in.at[pl.ds(0, N_CELL)],
                            out_hbm.at[pl.ds(base + N_CELL, N_CELL)])

    return k(xr)


def kernel(x):
    shape = x.shape
    out = _nms_sc(x.reshape(-1))
    return out.reshape(shape)


# speedup vs baseline: 59.5341x; 1.0257x over previous
import functools

import jax
import jax.numpy as jnp
from jax import lax
from jax.experimental import pallas as pl
from jax.experimental.pallas import tpu as pltpu
from jax.experimental.pallas import tpu_sc as plsc

N_CELL = 1000
NPAD = 1024
NCH = NPAD // 16
OFF = 112
WORK = OFF + NPAD + 112
BIGP = 1 << 29
MAXEV = 100
NSLICE = 16
UNROLL = 8


def _nms_sc(xr):
    mesh = plsc.VectorSubcoreMesh(
        core_axis_name="c", subcore_axis_name="s", num_cores=1)

    @functools.partial(
        pl.kernel,
        out_type=jax.ShapeDtypeStruct((NSLICE * 2 * N_CELL,), jnp.float32),
        mesh=mesh,
        compiler_params=pltpu.CompilerParams(needs_layout_passes=False),
        scratch_types=[
            pltpu.VMEM((NPAD,), jnp.float32),
            pltpu.VMEM((NPAD,), jnp.float32),
            pltpu.VMEM((WORK,), jnp.float32),
            pltpu.VMEM((WORK,), jnp.int32),
            pltpu.VMEM((WORK,), jnp.float32),
            pltpu.VMEM((WORK,), jnp.int32),
            pltpu.VMEM((WORK,), jnp.float32),
            pltpu.VMEM((WORK,), jnp.int32),
            pltpu.VMEM((NPAD,), jnp.int32),
            pltpu.VMEM((NPAD,), jnp.int32),
            pltpu.VMEM((NPAD,), jnp.int32),
            pltpu.VMEM((NPAD,), jnp.int32),
            pltpu.SemaphoreType.DMA,
        ],
    )
    def k(x_hbm, out_hbm, e_in, c1_in, ae, ap, be, bp, ce, cp,
          eb, mz, my, mx, c1_sem):
        wid = lax.axis_index("s")

        @pl.when(wid < NSLICE)
        def _():
            base = wid * (2 * N_CELL)
            c1_cp = pltpu.make_async_copy(
                x_hbm.at[pl.ds(base + N_CELL, N_CELL)],
                c1_in.at[pl.ds(0, N_CELL)], c1_sem)
            c1_cp.start()
            pltpu.sync_copy(x_hbm.at[pl.ds(base, N_CELL)],
                            e_in.at[pl.ds(0, N_CELL)])

            zerov = jnp.full((16,), 0, jnp.int32)

            falsev = jnp.full((16,), False, jnp.bool_)

            @plsc.parallel_loop(0, NCH, unroll=UNROLL, carry=falsev)
            def und0_vec(j, cnt):
                ids = j * 16 + lax.iota(jnp.int32, 16)
                sl = pl.ds(j * 16, 16)
                wsl = pl.ds(OFF + j * 16, 16)
                e = e_in[sl]
                valid = jnp.logical_and(ids < N_CELL, e != 0.0)
                ae[wsl] = jnp.where(valid, e, -1.0)
                ap[wsl] = jnp.where(valid, ids * 2, BIGP)
                eb[sl] = jnp.where(
                    valid, lax.bitcast_convert_type(e, jnp.int32), -1)
                z = lax.rem(ids, 10)
                y = lax.rem(lax.div(ids, 10), 10)
                xx = lax.div(ids, 100)
                one = jnp.full((16,), 1, jnp.int32)
                two = jnp.full((16,), 2, jnp.int32)
                mz[sl] = jnp.where(z > 0, one, zerov) | jnp.where(
                    z < 9, two, zerov)
                my[sl] = jnp.where(y > 0, one, zerov) | jnp.where(
                    y < 9, two, zerov)
                mx[sl] = jnp.where(xx > 0, one, zerov) | jnp.where(
                    xx < 9, two, zerov)
                return cnt | valid

            und0 = jnp.any(und0_vec)

            def pool(src_e, src_p, dst_e, dst_p, stride, m_ref):
                @plsc.parallel_loop(0, NCH, unroll=UNROLL)
                def _body(j):
                    b = OFF + j * 16
                    m = m_ref[pl.ds(j * 16, 16)]
                    ce_ = src_e[pl.ds(b, 16)]
                    cp_ = src_p[pl.ds(b, 16)]
                    le = src_e[pl.ds(b - stride, 16)]
                    lp = src_p[pl.ds(b - stride, 16)]
                    re_ = src_e[pl.ds(b + stride, 16)]
                    rp = src_p[pl.ds(b + stride, 16)]
                    mlo = (m & 1) != 0
                    mhi = (m & 2) != 0
                    le = jnp.where(mlo, le, -2.0)
                    lp = jnp.where(mlo, lp, BIGP)
                    re_ = jnp.where(mhi, re_, -2.0)
                    rp = jnp.where(mhi, rp, BIGP)
                    gt = jnp.logical_or(
                        ce_ > le, jnp.logical_and(ce_ == le, cp_ < lp))
                    me = jnp.where(gt, ce_, le)
                    mp = jnp.where(gt, cp_, lp)
                    gt2 = jnp.logical_or(
                        me > re_, jnp.logical_and(me == re_, mp < rp))
                    dst_e[pl.ds(b, 16)] = jnp.where(gt2, me, re_)
                    dst_p[pl.ds(b, 16)] = jnp.where(gt2, mp, rp)

            def round_body(_c):
                pool(ae, ap, be, bp, 1, mz)
                pool(be, bp, ce, cp, 10, my)

                @plsc.parallel_loop(0, NCH, unroll=UNROLL, carry=falsev)
                def rem_vec(j, cnt):
                    b = OFF + j * 16
                    sl = pl.ds(b, 16)
                    m = mx[pl.ds(j * 16, 16)]
                    ce_ = ce[sl]
                    cp_ = cp[sl]
                    le = ce[pl.ds(b - 100, 16)]
                    lp = cp[pl.ds(b - 100, 16)]
                    re_ = ce[pl.ds(b + 100, 16)]
                    rp = cp[pl.ds(b + 100, 16)]
                    mlo = (m & 1) != 0
                    mhi = (m & 2) != 0
                    le = jnp.where(mlo, le, -2.0)
                    lp = jnp.where(mlo, lp, BIGP)
                    re_ = jnp.where(mhi, re_, -2.0)
                    rp = jnp.where(mhi, rp, BIGP)
                    gt = jnp.logical_or(
                        ce_ > le, jnp.logical_and(ce_ == le, cp_ < lp))
                    me = jnp.where(gt, ce_, le)
                    mp = jnp.where(gt, cp_, lp)
                    gt2 = jnp.logical_or(
                        me > re_, jnp.logical_and(me == re_, mp < rp))
                    me = jnp.where(gt2, me, re_)
                    mp = jnp.where(gt2, mp, rp)
                    e = ae[sl]
                    p = ap[sl]
                    kb = (p & 1) == 1
                    und = jnp.logical_and(e > 0.0, jnp.logical_not(kb))
                    isself = mp == p
                    newk = jnp.logical_and(und, isself)
                    sup = und & ((mp & 1) == 1) & jnp.logical_not(isself)
                    ap[sl] = jnp.where(newk, p | 1, p)
                    ae[sl] = jnp.where(sup, -1.0, e)
                    rem = und & jnp.logical_not(newk) & jnp.logical_not(sup)
                    return cnt | rem

                return jnp.any(rem_vec)

            lax.while_loop(lambda c: c, round_body, und0)

            @plsc.parallel_loop(0, NCH, unroll=UNROLL, carry=zerov)
            def kept_vec(j, cnt):
                p = ap[pl.ds(OFF + j * 16, 16)]
                return cnt + (p & 1)

            kept_n = jnp.sum(kept_vec)

            @pl.when(kept_n > MAXEV)
            def _cap():
                def count_gt(t):
                    @plsc.parallel_loop(0, NCH, unroll=UNROLL, carry=zerov)
                    def cvec(j, cnt):
                        ebj = eb[pl.ds(j * 16, 16)]
                        return cnt + (ebj > t).astype(jnp.int32)
                    return jnp.sum(cvec)

                def bs(_i, lohi):
                    lo, hi = lohi
                    mid = lax.div(lo + hi, jnp.int32(2))
                    pred = count_gt(mid) < MAXEV
                    return (jnp.where(pred, lo, mid + 1),
                            jnp.where(pred, mid, hi))

                tau, _ = lax.fori_loop(
                    0, 30, bs, (jnp.int32(0), jnp.int32((1 << 30) - 1)))
                quota = MAXEV - count_gt(tau)

                def capb(j, carry):
                    b = pl.ds(OFF + j * 16, 16)
                    sl = pl.ds(j * 16, 16)
                    ebj = eb[sl]
                    tie = ebj == tau
                    tc = tie.astype(jnp.int32)
                    pfx = plsc.cumsum(tc)
                    surv = tie & ((carry + (pfx - tc)) < quota)
                    allow = (ebj > tau) | surv
                    p = ap[b]
                    ap[b] = jnp.where(allow, p, p & (~1))
                    return carry + jnp.max(pfx)

                lax.fori_loop(0, NCH, capb, jnp.int32(0))

            anyv = und0
            c1_cp.wait()

            @plsc.parallel_loop(0, NCH, unroll=UNROLL)
            def _ob(j):
                b = pl.ds(OFF + j * 16, 16)
                sl = pl.ds(j * 16, 16)
                keepm = jnp.logical_or((ap[b] & 1) == 1,
                                       jnp.logical_not(anyv))
                e_in[sl] = jnp.where(keepm, e_in[sl], 0.0)
                c1_in[sl] = jnp.where(keepm, c1_in[sl], 0.0)

            e_out = pltpu.make_async_copy(
                e_in.at[pl.ds(0, N_CELL)],
                out_hbm.at[pl.ds(base, N_CELL)], c1_sem)
            e_out.start()
            pltpu.sync_copy(c1_in.at[pl.ds(0, N_CELL)],
                            out_hbm.at[pl.ds(base + N_CELL, N_CELL)])
            e_out.wait()

    return k(xr)


def kernel(x):
    shape = x.shape
    out = _nms_sc(x.reshape(-1))
    return out.reshape(shape)
